# baseline (speedup 1.0000x reference)
import jax
import jax.numpy as jnp
from jax.experimental import pallas as pl
from jax.experimental.pallas import tpu as pltpu

N = 10000
NFEAT = 128
NHID = 64
NCLASS = 64
RB = 400
GRID = N // RB

_F32 = jnp.float32


def _dot(a, b):
    return jax.lax.dot_general(a, b, (((1,), (0,)), ((), ())),
                               preferred_element_type=_F32)


def _p1_body(adj_ref, x_ref, w_ref, b_ref, h0_ref, y_ref):
    @pl.when(pl.program_id(0) == 0)
    def _():
        y_ref[...] = _dot(x_ref[...], w_ref[...])

    h0_ref[...] = jnp.tanh(_dot(adj_ref[...], y_ref[...]) + b_ref[...])


def _p2_body(adj_ref, x_ref, h0_ref, w_ref, b_ref, h1_ref, y_ref):
    @pl.when(pl.program_id(0) == 0)
    def _():
        y_ref[...] = (_dot(x_ref[...], w_ref[:NFEAT, :])
                      + _dot(h0_ref[...], w_ref[NFEAT:, :]))

    h1_ref[...] = jnp.tanh(_dot(adj_ref[...], y_ref[...]) + b_ref[...])


def _p3_body(adj_ref, x_ref, h0_ref, h1_ref, w_ref, b_ref, out_ref, y_ref):
    @pl.when(pl.program_id(0) == 0)
    def _():
        y_ref[...] = (_dot(x_ref[...], w_ref[:NFEAT, :])
                      + _dot(h0_ref[...], w_ref[NFEAT:NFEAT + NHID, :])
                      + _dot(h1_ref[...], w_ref[NFEAT + NHID:, :]))

    out_ref[...] = _dot(adj_ref[...], y_ref[...]) + b_ref[...]


def _full(shape):
    return pl.BlockSpec(shape, lambda i: (0,) * len(shape))


def _rows(width):
    return pl.BlockSpec((RB, width), lambda i: (i, 0))


def kernel(x, adj, W0, b0, W1, b1, W_out, b_out):
    b0 = b0.reshape(1, NHID)
    b1 = b1.reshape(1, NHID)
    b_out = b_out.reshape(1, NCLASS)

    h0 = pl.pallas_call(
        _p1_body,
        grid=(GRID,),
        in_specs=[_rows(N), _full((N, NFEAT)), _full((NFEAT, NHID)),
                  _full((1, NHID))],
        out_specs=_rows(NHID),
        out_shape=jax.ShapeDtypeStruct((N, NHID), _F32),
        scratch_shapes=[pltpu.VMEM((N, NHID), _F32)],
    )(adj, x, W0, b0)

    h1 = pl.pallas_call(
        _p2_body,
        grid=(GRID,),
        in_specs=[_rows(N), _full((N, NFEAT)), _full((N, NHID)),
                  _full((NFEAT + NHID, NHID)), _full((1, NHID))],
        out_specs=_rows(NHID),
        out_shape=jax.ShapeDtypeStruct((N, NHID), _F32),
        scratch_shapes=[pltpu.VMEM((N, NHID), _F32)],
    )(adj, x, h0, W1, b1)

    out = pl.pallas_call(
        _p3_body,
        grid=(GRID,),
        in_specs=[_rows(N), _full((N, NFEAT)), _full((N, NHID)),
                  _full((N, NHID)), _full((NFEAT + 2 * NHID, NCLASS)),
                  _full((1, NCLASS))],
        out_specs=_rows(NCLASS),
        out_shape=jax.ShapeDtypeStruct((N, NCLASS), _F32),
        scratch_shapes=[pltpu.VMEM((N, NCLASS), _F32)],
    )(adj, x, h0, h1, W_out, b_out)

    return out


# speedup vs baseline: 1.0796x; 1.0796x over previous
import jax
import jax.numpy as jnp
from jax.experimental import pallas as pl
from jax.experimental.pallas import tpu as pltpu

N = 10000
NFEAT = 128
NHID = 64
NCLASS = 64
RB = 400
GRID = N // RB

_F32 = jnp.float32
_BF16 = jnp.bfloat16


def _dot(a, b):
    return jax.lax.dot_general(a, b, (((1,), (0,)), ((), ())),
                               preferred_element_type=_F32)


def _p1_body(adj_ref, x_ref, w_ref, b_ref, h0_ref, adjb_ref, y_ref):
    @pl.when(pl.program_id(0) == 0)
    def _():
        y_ref[...] = _dot(x_ref[...], w_ref[...]).astype(_BF16)

    ab = adj_ref[...].astype(_BF16)
    adjb_ref[...] = ab
    h0_ref[...] = jnp.tanh(_dot(ab, y_ref[...]) + b_ref[...])


def _p2_body(adjb_ref, x_ref, h0_ref, w_ref, b_ref, h1_ref, y_ref):
    @pl.when(pl.program_id(0) == 0)
    def _():
        y_ref[...] = (_dot(x_ref[...], w_ref[:NFEAT, :])
                      + _dot(h0_ref[...], w_ref[NFEAT:, :])).astype(_BF16)

    h1_ref[...] = jnp.tanh(_dot(adjb_ref[...], y_ref[...]) + b_ref[...])


def _p3_body(adjb_ref, x_ref, h0_ref, h1_ref, w_ref, b_ref, out_ref, y_ref):
    @pl.when(pl.program_id(0) == 0)
    def _():
        y_ref[...] = (_dot(x_ref[...], w_ref[:NFEAT, :])
                      + _dot(h0_ref[...], w_ref[NFEAT:NFEAT + NHID, :])
                      + _dot(h1_ref[...], w_ref[NFEAT + NHID:, :])).astype(_BF16)

    out_ref[...] = _dot(adjb_ref[...], y_ref[...]) + b_ref[...]


def _full(shape):
    return pl.BlockSpec(shape, lambda i: (0,) * len(shape))


def _rows(width):
    return pl.BlockSpec((RB, width), lambda i: (i, 0))


def kernel(x, adj, W0, b0, W1, b1, W_out, b_out):
    b0 = b0.reshape(1, NHID)
    b1 = b1.reshape(1, NHID)
    b_out = b_out.reshape(1, NCLASS)

    h0, adjb = pl.pallas_call(
        _p1_body,
        grid=(GRID,),
        in_specs=[_rows(N), _full((N, NFEAT)), _full((NFEAT, NHID)),
                  _full((1, NHID))],
        out_specs=[_rows(NHID), _rows(N)],
        out_shape=[jax.ShapeDtypeStruct((N, NHID), _F32),
                   jax.ShapeDtypeStruct((N, N), _BF16)],
        scratch_shapes=[pltpu.VMEM((N, NHID), _BF16)],
    )(adj, x, W0, b0)

    h1 = pl.pallas_call(
        _p2_body,
        grid=(GRID,),
        in_specs=[_rows(N), _full((N, NFEAT)), _full((N, NHID)),
                  _full((NFEAT + NHID, NHID)), _full((1, NHID))],
        out_specs=_rows(NHID),
        out_shape=jax.ShapeDtypeStruct((N, NHID), _F32),
        scratch_shapes=[pltpu.VMEM((N, NHID), _BF16)],
    )(adjb, x, h0, W1, b1)

    out = pl.pallas_call(
        _p3_body,
        grid=(GRID,),
        in_specs=[_rows(N), _full((N, NFEAT)), _full((N, NHID)),
                  _full((N, NHID)), _full((NFEAT + 2 * NHID, NCLASS)),
                  _full((1, NCLASS))],
        out_specs=_rows(NCLASS),
        out_shape=jax.ShapeDtypeStruct((N, NCLASS), _F32),
        scratch_shapes=[pltpu.VMEM((N, NCLASS), _BF16)],
    )(adjb, x, h0, h1, W_out, b_out)

    return out
